# baseline (speedup 1.0000x reference)
import functools

import jax
import jax.numpy as jnp
from jax import lax
from jax.experimental import pallas as pl
from jax.experimental.pallas import tpu as pltpu
from jax.experimental.pallas import tpu_sc as plsc

N_REGIMES_ = 3
N_COLS_ = 5
LANES_ = 16


def _take16(vec, idx):
    dnums = lax.GatherDimensionNumbers(
        offset_dims=(), collapsed_slice_dims=(0,), start_index_map=(0,)
    )
    return lax.gather(
        vec,
        idx[:, None],
        dnums,
        (1,),
        mode=lax.GatherScatterMode.PROMISE_IN_BOUNDS,
    )


@functools.lru_cache(maxsize=None)
def _build_sc_gather(batch: int):
    info = plsc.get_sparse_core_info()
    nc, ns = 1, 8
    nw = nc * ns
    assert batch % (nw * LANES_) == 0
    b_per_w = batch // nw
    mesh = plsc.VectorSubcoreMesh(
        core_axis_name="c", subcore_axis_name="s", num_cores=nc, num_subcores=ns
    )

    @functools.partial(
        pl.kernel,
        mesh=mesh,
        out_type=jax.ShapeDtypeStruct((N_COLS_, batch), jnp.float32),
        scratch_types=[
            pltpu.VMEM((b_per_w,), jnp.int32),
            pltpu.VMEM((LANES_,), jnp.float32),
            pltpu.VMEM((N_COLS_, b_per_w), jnp.float32),
            pltpu.SemaphoreType.DMA,
            pltpu.SemaphoreType.DMA,
        ],
    )
    def sc_gather(regime_hbm, table_hbm, out_hbm, idx_v, table_v, out_v, isem, tsem):
        wid = lax.axis_index("s") * nc + lax.axis_index("c")
        base = wid * b_per_w
        icp = pltpu.async_copy(regime_hbm.at[pl.ds(base, b_per_w)], idx_v, isem)
        pltpu.async_copy(
            table_hbm, table_v.at[pl.ds(0, N_REGIMES_ * N_COLS_)], tsem
        ).wait()
        table_reg = table_v[...]
        zero = jnp.zeros((LANES_,), jnp.int32)
        top = jnp.full((LANES_,), N_REGIMES_ - 1, jnp.int32)
        icp.wait()

        def body(u, carry):
            r = idx_v[pl.ds(u * LANES_, LANES_)]
            r5 = jnp.minimum(jnp.maximum(r, zero), top) * N_COLS_
            for j in range(N_COLS_):
                out_v[j, pl.ds(u * LANES_, LANES_)] = _take16(table_reg, r5 + j)
            return carry

        lax.fori_loop(0, b_per_w // LANES_, body, 0)
        pltpu.async_copy(
            out_v, out_hbm.at[:, pl.ds(base, b_per_w)], tsem
        ).wait()

    return sc_gather


def kernel(x, regime, regime_weights):
    del x
    batch = regime.shape[0]
    regime = regime.astype(jnp.int32)
    table = regime_weights.astype(jnp.float32).reshape(-1)
    out_t = _build_sc_gather(batch)(regime, table)
    return out_t.T


# speedup vs baseline: 1.0457x; 1.0457x over previous
import functools

import jax
import jax.numpy as jnp
from jax import lax
from jax.experimental import pallas as pl
from jax.experimental.pallas import tpu as pltpu
from jax.experimental.pallas import tpu_sc as plsc

N_REGIMES_ = 3
N_COLS_ = 5
LANES_ = 16


def _take16(vec, idx):
    dnums = lax.GatherDimensionNumbers(
        offset_dims=(), collapsed_slice_dims=(0,), start_index_map=(0,)
    )
    return lax.gather(
        vec,
        idx[:, None],
        dnums,
        (1,),
        mode=lax.GatherScatterMode.PROMISE_IN_BOUNDS,
    )


@functools.lru_cache(maxsize=None)
def _build_sc_gather(batch: int):
    info = plsc.get_sparse_core_info()
    nc, ns = 1, info.num_subcores
    nw = nc * ns
    assert batch % (nw * LANES_) == 0
    b_per_w = batch // nw
    mesh = plsc.VectorSubcoreMesh(
        core_axis_name="c", subcore_axis_name="s", num_cores=nc
    )

    @functools.partial(
        pl.kernel,
        mesh=mesh,
        out_type=jax.ShapeDtypeStruct((N_COLS_, batch), jnp.float32),
        scratch_types=[
            pltpu.VMEM((b_per_w,), jnp.int32),
            pltpu.VMEM((LANES_,), jnp.float32),
            pltpu.VMEM((N_COLS_, b_per_w), jnp.float32),
            pltpu.SemaphoreType.DMA,
            pltpu.SemaphoreType.DMA,
        ],
    )
    def sc_gather(regime_hbm, table_hbm, out_hbm, idx_v, table_v, out_v, isem, tsem):
        wid = lax.axis_index("s") * nc + lax.axis_index("c")
        base = wid * b_per_w
        icp = pltpu.async_copy(regime_hbm.at[pl.ds(base, b_per_w)], idx_v, isem)
        pltpu.async_copy(
            table_hbm, table_v.at[pl.ds(0, N_REGIMES_ * N_COLS_)], tsem
        ).wait()
        table_reg = table_v[...]
        zero = jnp.zeros((LANES_,), jnp.int32)
        top = jnp.full((LANES_,), N_REGIMES_ - 1, jnp.int32)
        lane5 = lax.iota(jnp.int32, LANES_) * N_COLS_
        top15 = jnp.full((LANES_,), LANES_ - 1, jnp.int32)
        cols = [
            _take16(table_reg, jnp.minimum(lane5 + j, top15))
            for j in range(N_COLS_)
        ]
        icp.wait()

        def step(u):
            r = idx_v[pl.ds(u * LANES_, LANES_)]
            rc = jnp.minimum(jnp.maximum(r, zero), top)
            for j in range(N_COLS_):
                out_v[j, pl.ds(u * LANES_, LANES_)] = _take16(cols[j], rc)

        def body(v, carry):
            step(v * 2)
            step(v * 2 + 1)
            return carry

        lax.fori_loop(0, b_per_w // (2 * LANES_), body, 0)
        pltpu.async_copy(
            out_v, out_hbm.at[:, pl.ds(base, b_per_w)], tsem
        ).wait()

    return sc_gather


def kernel(x, regime, regime_weights):
    del x
    batch = regime.shape[0]
    regime = regime.astype(jnp.int32)
    table = regime_weights.astype(jnp.float32).reshape(-1)
    out_t = _build_sc_gather(batch)(regime, table)
    return out_t.T


# speedup vs baseline: 1.0498x; 1.0039x over previous
import functools

import jax
import jax.numpy as jnp
from jax import lax
from jax.experimental import pallas as pl
from jax.experimental.pallas import tpu as pltpu
from jax.experimental.pallas import tpu_sc as plsc

N_REGIMES_ = 3
N_COLS_ = 5
LANES_ = 16


def _take16(vec, idx):
    dnums = lax.GatherDimensionNumbers(
        offset_dims=(), collapsed_slice_dims=(0,), start_index_map=(0,)
    )
    return lax.gather(
        vec,
        idx[:, None],
        dnums,
        (1,),
        mode=lax.GatherScatterMode.PROMISE_IN_BOUNDS,
    )


@functools.lru_cache(maxsize=None)
def _build_sc_gather(batch: int):
    info = plsc.get_sparse_core_info()
    nc, ns = 1, info.num_subcores
    nw = nc * ns
    assert batch % (nw * LANES_) == 0
    b_per_w = batch // nw
    mesh = plsc.VectorSubcoreMesh(
        core_axis_name="c", subcore_axis_name="s", num_cores=nc
    )

    @functools.partial(
        pl.kernel,
        mesh=mesh,
        out_type=jax.ShapeDtypeStruct((N_COLS_, batch), jnp.float32),
        scratch_types=[
            pltpu.VMEM((b_per_w,), jnp.int32),
            pltpu.VMEM((LANES_,), jnp.float32),
            pltpu.VMEM((N_COLS_, b_per_w), jnp.float32),
            pltpu.SemaphoreType.DMA,
            pltpu.SemaphoreType.DMA,
        ],
    )
    def sc_gather(regime_hbm, table_hbm, out_hbm, idx_v, table_v, out_v, isem, tsem):
        wid = lax.axis_index("s") * nc + lax.axis_index("c")
        base = wid * b_per_w
        icp = pltpu.async_copy(regime_hbm.at[pl.ds(base, b_per_w)], idx_v, isem)
        pltpu.async_copy(
            table_hbm, table_v.at[pl.ds(0, N_REGIMES_ * N_COLS_)], tsem
        ).wait()
        table_reg = table_v[...]
        zero = jnp.zeros((LANES_,), jnp.int32)
        top = jnp.full((LANES_,), N_REGIMES_ - 1, jnp.int32)
        lane5 = lax.iota(jnp.int32, LANES_) * N_COLS_
        top15 = jnp.full((LANES_,), LANES_ - 1, jnp.int32)
        cols = [
            _take16(table_reg, jnp.minimum(lane5 + j, top15))
            for j in range(N_COLS_)
        ]
        icp.wait()

        def step(u):
            r = idx_v[pl.ds(u * LANES_, LANES_)]
            rc = jnp.minimum(jnp.maximum(r, zero), top)
            for j in range(N_COLS_):
                out_v[j, pl.ds(u * LANES_, LANES_)] = _take16(cols[j], rc)

        def body(v, carry):
            step(v * 2)
            step(v * 2 + 1)
            return carry

        half = b_per_w // 2
        lax.fori_loop(0, half // (2 * LANES_), body, 0)
        first = pltpu.async_copy(
            out_v.at[:, pl.ds(0, half)],
            out_hbm.at[:, pl.ds(base, half)],
            isem,
        )
        lax.fori_loop(half // (2 * LANES_), b_per_w // (2 * LANES_), body, 0)
        pltpu.async_copy(
            out_v.at[:, pl.ds(half, half)],
            out_hbm.at[:, pl.ds(base + half, half)],
            tsem,
        ).wait()
        first.wait()

    return sc_gather


def kernel(x, regime, regime_weights):
    del x
    batch = regime.shape[0]
    regime = regime.astype(jnp.int32)
    table = regime_weights.astype(jnp.float32).reshape(-1)
    out_t = _build_sc_gather(batch)(regime, table)
    return out_t.T
